# baseline (speedup 1.0000x reference)
import functools

import jax
import jax.numpy as jnp
from jax import lax
from jax.experimental import pallas as pl
from jax.experimental.pallas import tpu as pltpu
from jax.experimental.pallas import tpu_sc as plsc

_INFO = plsc.get_sparse_core_info()
_NC = _INFO.num_cores
_NS = _INFO.num_subcores
_NW = _NC * _NS
_CHUNK = 128


def _make_gather(n_rows, d, batch):
    b_per_w = batch // _NW
    n_chunks = b_per_w // _CHUNK
    mesh = plsc.VectorSubcoreMesh(core_axis_name="c", subcore_axis_name="s")

    @functools.partial(
        pl.kernel,
        mesh=mesh,
        out_type=jax.ShapeDtypeStruct((batch, d), jnp.float32),
        scratch_types=[
            pltpu.VMEM((n_chunks, _CHUNK), jnp.int32),
            pltpu.VMEM((b_per_w, d), jnp.float32),
            pltpu.SemaphoreType.DMA,
        ],
        compiler_params=pltpu.CompilerParams(use_tc_tiling_on_sc=False),
    )
    def gather_kernel(idx_hbm, table_hbm, out_hbm, idx_v, rows_v, sem):
        wid = lax.axis_index("s") * _NC + lax.axis_index("c")
        pltpu.sync_copy(idx_hbm.at[wid], idx_v)
        copies = [
            pltpu.async_copy(
                table_hbm.at[idx_v.at[j]],
                rows_v.at[pl.ds(j * _CHUNK, _CHUNK)],
                sem,
            )
            for j in range(n_chunks)
        ]
        for c in copies:
            c.wait()
        pltpu.sync_copy(rows_v, out_hbm.at[pl.ds(wid * b_per_w, b_per_w)])

    return gather_kernel


def kernel(states_indices, table):
    batch = states_indices.shape[0]
    n_rows, d = table.shape
    idx3 = states_indices.astype(jnp.int32).reshape(_NW, batch // _NW // _CHUNK, _CHUNK)
    return _make_gather(n_rows, d, batch)(idx3, table)


# speedup vs baseline: 5.8334x; 5.8334x over previous
import functools

import jax
import jax.numpy as jnp
from jax import lax
from jax.experimental import pallas as pl
from jax.experimental.pallas import tpu as pltpu
from jax.experimental.pallas import tpu_sc as plsc

_INFO = plsc.get_sparse_core_info()
_NC = _INFO.num_cores
_NS = _INFO.num_subcores
_NW = _NC * _NS
_G = 16


def _make_gather(n_rows, d, batch):
    b_per_t = batch // _NW
    n_groups = b_per_t // _G
    mesh = plsc.VectorSubcoreMesh(core_axis_name="c", subcore_axis_name="s")

    @functools.partial(
        pl.kernel,
        mesh=mesh,
        out_type=jax.ShapeDtypeStruct((d, batch), jnp.float32),
        scratch_types=[
            pltpu.VMEM((b_per_t,), jnp.int32),
            pltpu.VMEM((2, d, _G * 128), jnp.float32),
            pltpu.VMEM((d, b_per_t), jnp.float32),
            pltpu.SemaphoreType.DMA,
            pltpu.SemaphoreType.DMA,
        ],
        compiler_params=pltpu.CompilerParams(
            use_tc_tiling_on_sc=True, needs_layout_passes=False
        ),
    )
    def gather_kernel(idx_hbm, tab_hbm, out_hbm, idx_v, buf, cols_v, sem0, sem1):
        wid = lax.axis_index("s") * _NC + lax.axis_index("c")
        base = wid * b_per_t
        pltpu.sync_copy(idx_hbm.at[pl.ds(base, b_per_t)], idx_v)
        feat = lax.iota(jnp.int32, _G)

        def fire(g, slot, sem):
            idx_vec = idx_v[pl.ds(g * _G, _G)]
            col_vec = lax.shift_right_logical(idx_vec, 7) * 128
            for k in range(_G):
                q = pl.multiple_of(col_vec[k], 128)
                pltpu.async_copy(
                    tab_hbm.at[:, pl.ds(q, 128)],
                    buf.at[slot].at[:, pl.ds(k * 128, 128)],
                    sem,
                )

        def drain(sem):
            pltpu.make_async_copy(
                tab_hbm.at[:, pl.ds(0, _G * 128)], buf.at[0], sem
            ).wait()

        def extract(g, slot):
            idx_vec = idx_v[pl.ds(g * _G, _G)]
            lane_vec = lax.bitwise_and(idx_vec, 127)
            slot_idx = jnp.full((_G,), slot, jnp.int32)
            for k in range(_G):
                pos = jnp.full((_G,), k * 128, jnp.int32) + jnp.broadcast_to(
                    lane_vec[k], (_G,)
                )
                vals = plsc.load_gather(buf, [slot_idx, feat, pos])
                plsc.store_scatter(
                    cols_v,
                    [feat, jnp.broadcast_to(g * _G + k, (_G,))],
                    vals,
                )

        fire(0, 0, sem0)

        def body(j, carry):
            fire(2 * j + 1, 1, sem1)
            drain(sem0)
            extract(2 * j, 0)
            fire(2 * j + 2, 0, sem0)
            drain(sem1)
            extract(2 * j + 1, 1)
            return carry

        lax.fori_loop(0, n_groups // 2 - 1, body, 0)
        fire(n_groups - 1, 1, sem1)
        drain(sem0)
        extract(n_groups - 2, 0)
        drain(sem1)
        extract(n_groups - 1, 1)

        pltpu.sync_copy(cols_v, out_hbm.at[:, pl.ds(base, b_per_t)])

    return gather_kernel


def kernel(states_indices, table):
    batch = states_indices.shape[0]
    n_rows, d = table.shape
    out_t = _make_gather(n_rows, d, batch)(states_indices.astype(jnp.int32), table.T)
    return out_t.T
